# baseline (speedup 1.0000x reference)
import jax
import jax.numpy as jnp
from jax import lax
from jax.experimental import pallas as pl
from jax.experimental.pallas import tpu as pltpu
from jax.experimental.pallas import tpu_sc as plsc

B, C, H, W = 4, 96, 224, 224
R = B * C
NC, NS = 2, 16
NW = NC * NS
WC = 112
NJ = WC // 16
HPW = H // NW
NCHUNK = HPW * (W // WC)


def _sc_body(x_hbm, m_hbm, mf_hbm, out1_hbm, om_hbm, xbuf, mbuf, selbuf, mfbuf):
    wid = lax.axis_index("s") * NC + lax.axis_index("c")
    pltpu.sync_copy(mf_hbm, mfbuf)
    mfv = mfbuf[...]

    def chunk(t, carry):
        h = wid * HPW + t // 2
        w0 = (t % 2) * WC
        pltpu.sync_copy(x_hbm.at[:, pl.ds(h, 1), pl.ds(w0, WC)], xbuf)
        pltpu.sync_copy(m_hbm.at[:, pl.ds(h, 1), pl.ds(w0, WC)], mbuf)

        for j in range(NJ):
            sl = pl.ds(j * 16, 16)
            zmax = jnp.full((16,), -jnp.inf, jnp.float32)
            for b in range(B):
                def cbody(c, acc, b=b, sl=sl):
                    row = b * C + c
                    return acc + (mfv * xbuf[row, 0, sl] + mbuf[row, 0, sl])
                s = lax.fori_loop(0, C, cbody, jnp.zeros((16,), jnp.float32))
                zmax = jnp.maximum(zmax, s)
            selbuf[sl] = jnp.where(zmax > 1.0, 0.0, mfv)

        def rbody(row, c2):
            for j in range(NJ):
                sl = pl.ds(j * 16, 16)
                xv = xbuf[row, 0, sl]
                om = mbuf[row, 0, sl] + selbuf[sl] * xv
                mbuf[row, 0, sl] = om
                xbuf[row, 0, sl] = om * xv
            return c2
        lax.fori_loop(0, R, rbody, 0)

        pltpu.sync_copy(xbuf, out1_hbm.at[:, pl.ds(h, 1), pl.ds(w0, WC)])
        pltpu.sync_copy(mbuf, om_hbm.at[:, pl.ds(h, 1), pl.ds(w0, WC)])
        return carry

    lax.fori_loop(0, NCHUNK, chunk, 0)


def kernel(x, mask, m, v):
    del v
    xf = x.reshape(R, H, W)
    mkf = mask.reshape(R, H, W)
    mvec = jnp.full((16,), m, dtype=jnp.float32)
    mesh = plsc.VectorSubcoreMesh(core_axis_name="c", subcore_axis_name="s")
    out1, om = pl.kernel(
        _sc_body,
        out_type=(
            jax.ShapeDtypeStruct((R, H, W), jnp.float32),
            jax.ShapeDtypeStruct((R, H, W), jnp.float32),
        ),
        mesh=mesh,
        compiler_params=pltpu.CompilerParams(use_tc_tiling_on_sc=False),
        scratch_types=[
            pltpu.VMEM((R, 1, WC), jnp.float32),
            pltpu.VMEM((R, 1, WC), jnp.float32),
            pltpu.VMEM((WC,), jnp.float32),
            pltpu.VMEM((16,), jnp.float32),
        ],
    )(xf, mkf, mvec)
    return out1.reshape(B, C, H, W), om.reshape(B, C, H, W)


# speedup vs baseline: 1.1273x; 1.1273x over previous
import jax
import jax.numpy as jnp
from jax import lax
from jax.experimental import pallas as pl
from jax.experimental.pallas import tpu as pltpu
from jax.experimental.pallas import tpu_sc as plsc

B, C, H, W = 4, 96, 224, 224
R = B * C
NC, NS = 2, 16
NW = NC * NS
WC = 112
NJ = WC // 16
HPW = H // NW
NCHUNK = HPW * (W // WC)


def _sc_body(x_hbm, m_hbm, mf_hbm, out1_hbm, om_hbm, xbuf, mbuf, selbuf, mfbuf,
             sem_a, sem_b):
    wid = lax.axis_index("s") * NC + lax.axis_index("c")
    pltpu.sync_copy(mf_hbm, mfbuf)
    mfv = mfbuf[...]

    def chunk(t, carry):
        h = wid * HPW + t // 2
        w0 = (t % 2) * WC
        din_x = pltpu.async_copy(x_hbm.at[:, pl.ds(h, 1), pl.ds(w0, WC)],
                                 xbuf, sem_a)
        din_m = pltpu.async_copy(m_hbm.at[:, pl.ds(h, 1), pl.ds(w0, WC)],
                                 mbuf, sem_b)
        din_x.wait()
        din_m.wait()

        for j in range(NJ):
            sl = pl.ds(j * 16, 16)
            zmax = jnp.full((16,), -jnp.inf, jnp.float32)
            for b in range(B):
                def cbody(c, acc, b=b, sl=sl):
                    row = b * C + c
                    return acc + (mfv * xbuf[row, 0, sl] + mbuf[row, 0, sl])
                s = lax.fori_loop(0, C, cbody, jnp.zeros((16,), jnp.float32),
                                  unroll=8)
                zmax = jnp.maximum(zmax, s)
            selbuf[sl] = jnp.where(zmax > 1.0, 0.0, mfv)

        def rbody(row, c2):
            for j in range(NJ):
                sl = pl.ds(j * 16, 16)
                xv = xbuf[row, 0, sl]
                om = mbuf[row, 0, sl] + selbuf[sl] * xv
                mbuf[row, 0, sl] = om
                xbuf[row, 0, sl] = om * xv
            return c2
        lax.fori_loop(0, R, rbody, 0, unroll=4)

        dout_x = pltpu.async_copy(xbuf, out1_hbm.at[:, pl.ds(h, 1), pl.ds(w0, WC)],
                                  sem_a)
        dout_m = pltpu.async_copy(mbuf, om_hbm.at[:, pl.ds(h, 1), pl.ds(w0, WC)],
                                  sem_b)
        dout_x.wait()
        dout_m.wait()
        return carry

    lax.fori_loop(0, NCHUNK, chunk, 0)


def kernel(x, mask, m, v):
    del v
    xf = x.reshape(R, H, W)
    mkf = mask.reshape(R, H, W)
    mvec = jnp.full((16,), m, dtype=jnp.float32)
    mesh = plsc.VectorSubcoreMesh(core_axis_name="c", subcore_axis_name="s")
    out1, om = pl.kernel(
        _sc_body,
        out_type=(
            jax.ShapeDtypeStruct((R, H, W), jnp.float32),
            jax.ShapeDtypeStruct((R, H, W), jnp.float32),
        ),
        mesh=mesh,
        compiler_params=pltpu.CompilerParams(use_tc_tiling_on_sc=False),
        scratch_types=[
            pltpu.VMEM((R, 1, WC), jnp.float32),
            pltpu.VMEM((R, 1, WC), jnp.float32),
            pltpu.VMEM((WC,), jnp.float32),
            pltpu.VMEM((16,), jnp.float32),
            pltpu.SemaphoreType.DMA,
            pltpu.SemaphoreType.DMA,
        ],
    )(xf, mkf, mvec)
    return out1.reshape(B, C, H, W), om.reshape(B, C, H, W)
